# Initial kernel scaffold; baseline (speedup 1.0000x reference)
import jax
import jax.numpy as jnp
from jax.experimental import pallas as pl


def kernel(triangles, points):
    raise NotImplementedError("write your pallas kernel here")


# speedup vs baseline: 58.1575x; 58.1575x over previous
import functools

import jax
import jax.numpy as jnp
from jax.experimental import pallas as pl

_EPS = 1e-12
_BIG_I32 = 2**31 - 1


def _safe_div(a, b):
    b_safe = jnp.where(jnp.abs(b) < _EPS, jnp.where(b < 0, -_EPS, _EPS), b)
    return a / b_safe


def _nearest_kernel(tri_ref, pts_ref, d_ref, cp_ref, idx_ref, bc_ref, *, F, FT, CQ):
    px = pts_ref[:, 0:1]
    py = pts_ref[:, 1:2]
    pz = pts_ref[:, 2:3]

    best_d = jnp.full((CQ, 1), jnp.inf, jnp.float32)
    best_idx = jnp.zeros((CQ, 1), jnp.int32)
    best_u = jnp.zeros((CQ, 1), jnp.float32)
    best_v = jnp.zeros((CQ, 1), jnp.float32)
    best_w = jnp.zeros((CQ, 1), jnp.float32)
    best_cx = jnp.zeros((CQ, 1), jnp.float32)
    best_cy = jnp.zeros((CQ, 1), jnp.float32)
    best_cz = jnp.zeros((CQ, 1), jnp.float32)

    lane = jax.lax.broadcasted_iota(jnp.int32, (1, FT), 1)

    for t in range(F // FT):
        s = slice(t * FT, (t + 1) * FT)
        ax = tri_ref[0:1, s]
        ay = tri_ref[1:2, s]
        az = tri_ref[2:3, s]
        bx = tri_ref[3:4, s]
        by = tri_ref[4:5, s]
        bz = tri_ref[5:6, s]
        cx = tri_ref[6:7, s]
        cy = tri_ref[7:8, s]
        cz = tri_ref[8:9, s]

        abx = bx - ax
        aby = by - ay
        abz = bz - az
        acx = cx - ax
        acy = cy - ay
        acz = cz - az

        apx = px - ax
        apy = py - ay
        apz = pz - az
        d1 = abx * apx + aby * apy + abz * apz
        d2 = acx * apx + acy * apy + acz * apz

        bpx = px - bx
        bpy = py - by
        bpz = pz - bz
        d3 = abx * bpx + aby * bpy + abz * bpz
        d4 = acx * bpx + acy * bpy + acz * bpz

        cpx = px - cx
        cpy = py - cy
        cpz = pz - cz
        d5 = abx * cpx + aby * cpy + abz * cpz
        d6 = acx * cpx + acy * cpy + acz * cpz

        vc = d1 * d4 - d3 * d2
        vb = d5 * d2 - d1 * d6
        va = d3 * d6 - d5 * d4

        v_ab = _safe_div(d1, d1 - d3)
        w_ac = _safe_div(d2, d2 - d6)
        w_bc = _safe_div(d4 - d3, (d4 - d3) + (d5 - d6))
        denom = _safe_div(jnp.ones_like(va), va + vb + vc)
        v_in = vb * denom
        w_in = vc * denom

        cond_a = (d1 <= 0) & (d2 <= 0)
        cond_b = (d3 >= 0) & (d4 <= d3)
        cond_ab = (vc <= 0) & (d1 >= 0) & (d3 <= 0)
        cond_c = (d6 >= 0) & (d5 <= d6)
        cond_ac = (vb <= 0) & (d2 >= 0) & (d6 <= 0)
        cond_bc = (va <= 0) & ((d4 - d3) >= 0) & ((d5 - d6) >= 0)

        u = 1.0 - v_in - w_in
        v = v_in
        w = w_in

        zero = jnp.zeros_like(u)
        u = jnp.where(cond_bc, zero, u)
        v = jnp.where(cond_bc, 1.0 - w_bc, v)
        w = jnp.where(cond_bc, w_bc, w)
        u = jnp.where(cond_ac, 1.0 - w_ac, u)
        v = jnp.where(cond_ac, zero, v)
        w = jnp.where(cond_ac, w_ac, w)
        u = jnp.where(cond_c, 0.0, u)
        v = jnp.where(cond_c, 0.0, v)
        w = jnp.where(cond_c, 1.0, w)
        u = jnp.where(cond_ab, 1.0 - v_ab, u)
        v = jnp.where(cond_ab, v_ab, v)
        w = jnp.where(cond_ab, zero, w)
        u = jnp.where(cond_b, 0.0, u)
        v = jnp.where(cond_b, 1.0, v)
        w = jnp.where(cond_b, 0.0, w)
        u = jnp.where(cond_a, 1.0, u)
        v = jnp.where(cond_a, 0.0, v)
        w = jnp.where(cond_a, 0.0, w)

        clx = u * ax + v * bx + w * cx
        cly = u * ay + v * by + w * cy
        clz = u * az + v * bz + w * cz
        dx = px - clx
        dy = py - cly
        dz = pz - clz
        dist = dx * dx + dy * dy + dz * dz

        dmin = jnp.min(dist, axis=1, keepdims=True)
        at_min = dist == dmin
        idx_t = jnp.min(jnp.where(at_min, lane, _BIG_I32), axis=1, keepdims=True)
        sel = lane == idx_t

        def pick(val):
            return jnp.sum(jnp.where(sel, val, 0.0), axis=1, keepdims=True)

        u_t, v_t, w_t = pick(u), pick(v), pick(w)
        cx_t, cy_t, cz_t = pick(clx), pick(cly), pick(clz)

        better = dmin < best_d
        best_d = jnp.where(better, dmin, best_d)
        best_idx = jnp.where(better, idx_t + t * FT, best_idx)
        best_u = jnp.where(better, u_t, best_u)
        best_v = jnp.where(better, v_t, best_v)
        best_w = jnp.where(better, w_t, best_w)
        best_cx = jnp.where(better, cx_t, best_cx)
        best_cy = jnp.where(better, cy_t, best_cy)
        best_cz = jnp.where(better, cz_t, best_cz)

    d_ref[:, :] = best_d
    idx_ref[:, :] = best_idx
    cp_ref[:, :] = jnp.concatenate([best_cx, best_cy, best_cz], axis=1)
    bc_ref[:, :] = jnp.concatenate([best_u, best_v, best_w], axis=1)


def _query_one(tris, pts, CQ=256, FT=512):
    F = tris.shape[0]
    Q = pts.shape[0]
    tri9 = tris.reshape(F, 9).T

    out_shape = [
        jax.ShapeDtypeStruct((Q, 1), jnp.float32),
        jax.ShapeDtypeStruct((Q, 3), jnp.float32),
        jax.ShapeDtypeStruct((Q, 1), jnp.int32),
        jax.ShapeDtypeStruct((Q, 3), jnp.float32),
    ]
    grid = (Q // CQ,)
    d, cp, idx, bc = pl.pallas_call(
        functools.partial(_nearest_kernel, F=F, FT=FT, CQ=CQ),
        grid=grid,
        in_specs=[
            pl.BlockSpec((9, F), lambda i: (0, 0)),
            pl.BlockSpec((CQ, 3), lambda i: (i, 0)),
        ],
        out_specs=[
            pl.BlockSpec((CQ, 1), lambda i: (i, 0)),
            pl.BlockSpec((CQ, 3), lambda i: (i, 0)),
            pl.BlockSpec((CQ, 1), lambda i: (i, 0)),
            pl.BlockSpec((CQ, 3), lambda i: (i, 0)),
        ],
        out_shape=out_shape,
    )(tri9, pts)
    return d[:, 0], cp, idx[:, 0], bc


def kernel(triangles, points):
    B = triangles.shape[0]
    ds, cps, idxs, bcs = [], [], [], []
    for b in range(B):
        d, cp, idx, bc = _query_one(triangles[b], points[b])
        ds.append(d)
        cps.append(cp)
        idxs.append(idx)
        bcs.append(bc)
    distances = jnp.stack(ds)
    closest_points = jnp.stack(cps)
    closest_faces = jnp.stack(idxs).astype(jnp.int64)
    closest_bcs = jnp.stack(bcs)
    return distances, closest_points, closest_faces, closest_bcs


# speedup vs baseline: 60.3946x; 1.0385x over previous
import functools

import jax
import jax.numpy as jnp
from jax import lax
from jax.experimental import pallas as pl
from jax.experimental.pallas import tpu as pltpu
from jax.experimental.pallas import tpu_sc as plsc

_EPS = 1e-12
_BIG_I32 = 2**31 - 1

_Q_SC = 512
_NW = 32


def _safe_div(a, b):
    b_safe = jnp.where(jnp.abs(b) < _EPS, jnp.where(b < 0, -_EPS, _EPS), b)
    return a / b_safe


def _ericson(px, py, pz, ax, ay, az, bx, by, bz, cx, cy, cz):
    abx = bx - ax
    aby = by - ay
    abz = bz - az
    acx = cx - ax
    acy = cy - ay
    acz = cz - az

    apx = px - ax
    apy = py - ay
    apz = pz - az
    d1 = abx * apx + aby * apy + abz * apz
    d2 = acx * apx + acy * apy + acz * apz

    bpx = px - bx
    bpy = py - by
    bpz = pz - bz
    d3 = abx * bpx + aby * bpy + abz * bpz
    d4 = acx * bpx + acy * bpy + acz * bpz

    cpx = px - cx
    cpy = py - cy
    cpz = pz - cz
    d5 = abx * cpx + aby * cpy + abz * cpz
    d6 = acx * cpx + acy * cpy + acz * cpz

    vc = d1 * d4 - d3 * d2
    vb = d5 * d2 - d1 * d6
    va = d3 * d6 - d5 * d4

    v_ab = _safe_div(d1, d1 - d3)
    w_ac = _safe_div(d2, d2 - d6)
    w_bc = _safe_div(d4 - d3, (d4 - d3) + (d5 - d6))
    denom = _safe_div(jnp.ones_like(va), va + vb + vc)
    v_in = vb * denom
    w_in = vc * denom

    cond_a = (d1 <= 0) & (d2 <= 0)
    cond_b = (d3 >= 0) & (d4 <= d3)
    cond_ab = (vc <= 0) & (d1 >= 0) & (d3 <= 0)
    cond_c = (d6 >= 0) & (d5 <= d6)
    cond_ac = (vb <= 0) & (d2 >= 0) & (d6 <= 0)
    cond_bc = (va <= 0) & ((d4 - d3) >= 0) & ((d5 - d6) >= 0)

    u = 1.0 - v_in - w_in
    v = v_in
    w = w_in

    zero = jnp.zeros_like(u)
    one = jnp.ones_like(u)
    u = jnp.where(cond_bc, zero, u)
    v = jnp.where(cond_bc, 1.0 - w_bc, v)
    w = jnp.where(cond_bc, w_bc, w)
    u = jnp.where(cond_ac, 1.0 - w_ac, u)
    v = jnp.where(cond_ac, zero, v)
    w = jnp.where(cond_ac, w_ac, w)
    u = jnp.where(cond_c, zero, u)
    v = jnp.where(cond_c, zero, v)
    w = jnp.where(cond_c, one, w)
    u = jnp.where(cond_ab, 1.0 - v_ab, u)
    v = jnp.where(cond_ab, v_ab, v)
    w = jnp.where(cond_ab, zero, w)
    u = jnp.where(cond_b, zero, u)
    v = jnp.where(cond_b, one, v)
    w = jnp.where(cond_b, zero, w)
    u = jnp.where(cond_a, one, u)
    v = jnp.where(cond_a, zero, v)
    w = jnp.where(cond_a, zero, w)

    clx = u * ax + v * bx + w * cx
    cly = u * ay + v * by + w * cy
    clz = u * az + v * bz + w * cz
    dx = px - clx
    dy = py - cly
    dz = pz - clz
    dist = dx * dx + dy * dy + dz * dz
    return dist, u, v, w, clx, cly, clz


def _tc_kernel(tri_ref, pts_ref, d_ref, cp_ref, idx_ref, bc_ref, *, F, FT, CQ):
    px = pts_ref[:, 0:1]
    py = pts_ref[:, 1:2]
    pz = pts_ref[:, 2:3]

    best_d = jnp.full((CQ, 1), jnp.inf, jnp.float32)
    best_idx = jnp.zeros((CQ, 1), jnp.int32)
    best_u = jnp.zeros((CQ, 1), jnp.float32)
    best_v = jnp.zeros((CQ, 1), jnp.float32)
    best_w = jnp.zeros((CQ, 1), jnp.float32)
    best_cx = jnp.zeros((CQ, 1), jnp.float32)
    best_cy = jnp.zeros((CQ, 1), jnp.float32)
    best_cz = jnp.zeros((CQ, 1), jnp.float32)

    lane = lax.broadcasted_iota(jnp.int32, (1, FT), 1)

    for t in range(F // FT):
        s = slice(t * FT, (t + 1) * FT)
        dist, u, v, w, clx, cly, clz = _ericson(
            px, py, pz,
            tri_ref[0:1, s], tri_ref[1:2, s], tri_ref[2:3, s],
            tri_ref[3:4, s], tri_ref[4:5, s], tri_ref[5:6, s],
            tri_ref[6:7, s], tri_ref[7:8, s], tri_ref[8:9, s],
        )

        dmin = jnp.min(dist, axis=1, keepdims=True)
        at_min = dist == dmin
        idx_t = jnp.min(jnp.where(at_min, lane, _BIG_I32), axis=1, keepdims=True)
        sel = lane == idx_t

        def pick(val):
            return jnp.sum(jnp.where(sel, val, 0.0), axis=1, keepdims=True)

        u_t, v_t, w_t = pick(u), pick(v), pick(w)
        cx_t, cy_t, cz_t = pick(clx), pick(cly), pick(clz)

        better = dmin < best_d
        best_d = jnp.where(better, dmin, best_d)
        best_idx = jnp.where(better, idx_t + t * FT, best_idx)
        best_u = jnp.where(better, u_t, best_u)
        best_v = jnp.where(better, v_t, best_v)
        best_w = jnp.where(better, w_t, best_w)
        best_cx = jnp.where(better, cx_t, best_cx)
        best_cy = jnp.where(better, cy_t, best_cy)
        best_cz = jnp.where(better, cz_t, best_cz)

    d_ref[:, :] = best_d
    idx_ref[:, :] = best_idx
    cp_ref[:, :] = jnp.concatenate([best_cx, best_cy, best_cz], axis=1)
    bc_ref[:, :] = jnp.concatenate([best_u, best_v, best_w], axis=1)


def _tc_query(tri9, pts, CQ=256, FT=512):
    F = tri9.shape[1]
    Q = pts.shape[0]
    out_shape = [
        jax.ShapeDtypeStruct((Q, 1), jnp.float32),
        jax.ShapeDtypeStruct((Q, 3), jnp.float32),
        jax.ShapeDtypeStruct((Q, 1), jnp.int32),
        jax.ShapeDtypeStruct((Q, 3), jnp.float32),
    ]
    grid = (Q // CQ,)
    d, cp, idx, bc = pl.pallas_call(
        functools.partial(_tc_kernel, F=F, FT=FT, CQ=CQ),
        grid=grid,
        in_specs=[
            pl.BlockSpec((9, F), lambda i: (0, 0)),
            pl.BlockSpec((CQ, 3), lambda i: (i, 0)),
        ],
        out_specs=[
            pl.BlockSpec((CQ, 1), lambda i: (i, 0)),
            pl.BlockSpec((CQ, 3), lambda i: (i, 0)),
            pl.BlockSpec((CQ, 1), lambda i: (i, 0)),
            pl.BlockSpec((CQ, 3), lambda i: (i, 0)),
        ],
        out_shape=out_shape,
    )(tri9, pts)
    return d[:, 0], cp, idx[:, 0], bc


def _take16(x, perm):
    return lax.gather(
        x, perm[:, None],
        dimension_numbers=lax.GatherDimensionNumbers(
            offset_dims=(), collapsed_slice_dims=(0,), start_index_map=(0,)),
        slice_sizes=(1,),
        mode=lax.GatherScatterMode.PROMISE_IN_BOUNDS,
    )


def _sc_body(tri_hbm, px_hbm, py_hbm, pz_hbm,
             d_hbm, i_hbm, u_hbm, v_hbm, w_hbm, cx_hbm, cy_hbm, cz_hbm,
             tri_v, px_v, py_v, pz_v,
             do_v, io_v, uo_v, vo_v, wo_v, cxo_v, cyo_v, czo_v,
             *, F, CH):
    wid = lax.axis_index("s") * 2 + lax.axis_index("c")
    base = wid * CH

    pltpu.sync_copy(tri_hbm, tri_v)
    pltpu.sync_copy(px_hbm.at[pl.ds(base, CH)], px_v)
    pltpu.sync_copy(py_hbm.at[pl.ds(base, CH)], py_v)
    pltpu.sync_copy(pz_hbm.at[pl.ds(base, CH)], pz_v)

    lane = lax.iota(jnp.int32, 16)

    def point_body(i, carry):
        iv = jnp.full((16,), i, jnp.int32)
        px = plsc.load_gather(px_v, [iv])
        py = plsc.load_gather(py_v, [iv])
        pz = plsc.load_gather(pz_v, [iv])

        init = (
            jnp.full((16,), jnp.inf, jnp.float32),
            jnp.zeros((16,), jnp.int32),
            jnp.zeros((16,), jnp.float32),
            jnp.zeros((16,), jnp.float32),
            jnp.zeros((16,), jnp.float32),
            jnp.zeros((16,), jnp.float32),
            jnp.zeros((16,), jnp.float32),
            jnp.zeros((16,), jnp.float32),
        )

        def tri_body(t, c):
            bd, bi, bu, bv, bw, bx_, by_, bz_ = c
            o = t * 16
            dist, u, v, w, clx, cly, clz = _ericson(
                px, py, pz,
                tri_v[pl.ds(o, 16)],
                tri_v[pl.ds(F + o, 16)],
                tri_v[pl.ds(2 * F + o, 16)],
                tri_v[pl.ds(3 * F + o, 16)],
                tri_v[pl.ds(4 * F + o, 16)],
                tri_v[pl.ds(5 * F + o, 16)],
                tri_v[pl.ds(6 * F + o, 16)],
                tri_v[pl.ds(7 * F + o, 16)],
                tri_v[pl.ds(8 * F + o, 16)],
            )
            tidx = lane + o
            better = dist < bd
            return (
                jnp.where(better, dist, bd),
                jnp.where(better, tidx, bi),
                jnp.where(better, u, bu),
                jnp.where(better, v, bv),
                jnp.where(better, w, bw),
                jnp.where(better, clx, bx_),
                jnp.where(better, cly, by_),
                jnp.where(better, clz, bz_),
            )

        bd, bi, bu, bv, bw, bcx, bcy, bcz = lax.fori_loop(
            0, F // 16, tri_body, init)

        for s in (8, 4, 2, 1):
            perm = lane ^ s
            d2 = _take16(bd, perm)
            i2 = _take16(bi, perm)
            u2 = _take16(bu, perm)
            v2 = _take16(bv, perm)
            w2 = _take16(bw, perm)
            x2 = _take16(bcx, perm)
            y2 = _take16(bcy, perm)
            z2 = _take16(bcz, perm)
            better = (d2 < bd) | ((d2 == bd) & (i2 < bi))
            bd = jnp.where(better, d2, bd)
            bi = jnp.where(better, i2, bi)
            bu = jnp.where(better, u2, bu)
            bv = jnp.where(better, v2, bv)
            bw = jnp.where(better, w2, bw)
            bcx = jnp.where(better, x2, bcx)
            bcy = jnp.where(better, y2, bcy)
            bcz = jnp.where(better, z2, bcz)

        lane0 = lane == jnp.zeros((16,), jnp.int32)
        plsc.store_scatter(do_v, [iv], bd, mask=lane0)
        plsc.store_scatter(io_v, [iv], bi, mask=lane0)
        plsc.store_scatter(uo_v, [iv], bu, mask=lane0)
        plsc.store_scatter(vo_v, [iv], bv, mask=lane0)
        plsc.store_scatter(wo_v, [iv], bw, mask=lane0)
        plsc.store_scatter(cxo_v, [iv], bcx, mask=lane0)
        plsc.store_scatter(cyo_v, [iv], bcy, mask=lane0)
        plsc.store_scatter(czo_v, [iv], bcz, mask=lane0)
        return carry

    lax.fori_loop(0, CH, point_body, 0)

    pltpu.sync_copy(do_v, d_hbm.at[pl.ds(base, CH)])
    pltpu.sync_copy(io_v, i_hbm.at[pl.ds(base, CH)])
    pltpu.sync_copy(uo_v, u_hbm.at[pl.ds(base, CH)])
    pltpu.sync_copy(vo_v, v_hbm.at[pl.ds(base, CH)])
    pltpu.sync_copy(wo_v, w_hbm.at[pl.ds(base, CH)])
    pltpu.sync_copy(cxo_v, cx_hbm.at[pl.ds(base, CH)])
    pltpu.sync_copy(cyo_v, cy_hbm.at[pl.ds(base, CH)])
    pltpu.sync_copy(czo_v, cz_hbm.at[pl.ds(base, CH)])


def _sc_query(tri9, pts):
    F = tri9.shape[1]
    Q = pts.shape[0]
    CH = Q // _NW
    tri_flat = tri9.reshape(9 * F)
    px, py, pz = pts[:, 0], pts[:, 1], pts[:, 2]

    f32 = jnp.float32
    call = pl.kernel(
        functools.partial(_sc_body, F=F, CH=CH),
        out_type=[
            jax.ShapeDtypeStruct((Q,), f32),
            jax.ShapeDtypeStruct((Q,), jnp.int32),
            jax.ShapeDtypeStruct((Q,), f32),
            jax.ShapeDtypeStruct((Q,), f32),
            jax.ShapeDtypeStruct((Q,), f32),
            jax.ShapeDtypeStruct((Q,), f32),
            jax.ShapeDtypeStruct((Q,), f32),
            jax.ShapeDtypeStruct((Q,), f32),
        ],
        mesh=plsc.VectorSubcoreMesh(core_axis_name="c", subcore_axis_name="s", num_cores=2),
        compiler_params=pltpu.CompilerParams(needs_layout_passes=False),
        scratch_types=[
            pltpu.VMEM((9 * F,), f32),
            pltpu.VMEM((CH,), f32),
            pltpu.VMEM((CH,), f32),
            pltpu.VMEM((CH,), f32),
            pltpu.VMEM((CH,), f32),
            pltpu.VMEM((CH,), jnp.int32),
            pltpu.VMEM((CH,), f32),
            pltpu.VMEM((CH,), f32),
            pltpu.VMEM((CH,), f32),
            pltpu.VMEM((CH,), f32),
            pltpu.VMEM((CH,), f32),
            pltpu.VMEM((CH,), f32),
        ],
    )
    d, idx, u, v, w, cx, cy, cz = call(tri_flat, px, py, pz)
    cp = jnp.stack([cx, cy, cz], axis=1)
    bc = jnp.stack([u, v, w], axis=1)
    return d, cp, idx, bc


def _query_one(tris, pts):
    F = tris.shape[0]
    Q = pts.shape[0]
    tri9 = tris.reshape(F, 9).T

    q_sc = _Q_SC
    q_tc = Q - q_sc
    d_tc, cp_tc, idx_tc, bc_tc = _tc_query(tri9, pts[:q_tc])
    d_sc, cp_sc, idx_sc, bc_sc = _sc_query(tri9, pts[q_tc:])
    d = jnp.concatenate([d_tc, d_sc])
    cp = jnp.concatenate([cp_tc, cp_sc])
    idx = jnp.concatenate([idx_tc, idx_sc])
    bc = jnp.concatenate([bc_tc, bc_sc])
    return d, cp, idx, bc


def kernel(triangles, points):
    B = triangles.shape[0]
    ds, cps, idxs, bcs = [], [], [], []
    for b in range(B):
        d, cp, idx, bc = _query_one(triangles[b], points[b])
        ds.append(d)
        cps.append(cp)
        idxs.append(idx)
        bcs.append(bc)
    distances = jnp.stack(ds)
    closest_points = jnp.stack(cps)
    closest_faces = jnp.stack(idxs).astype(jnp.int64)
    closest_bcs = jnp.stack(bcs)
    return distances, closest_points, closest_faces, closest_bcs


# speedup vs baseline: 74.6113x; 1.2354x over previous
import functools

import jax
import jax.numpy as jnp
from jax import lax
from jax.experimental import pallas as pl
from jax.experimental.pallas import tpu as pltpu
from jax.experimental.pallas import tpu_sc as plsc

_EPS = 1e-12
_BIG_I32 = 2**31 - 1

_Q_SC = 2048
_NW = 32


def _safe_div(a, b):
    b_safe = jnp.where(jnp.abs(b) < _EPS, jnp.where(b < 0, -_EPS, _EPS), b)
    return a / b_safe


def _ericson(px, py, pz, ax, ay, az, bx, by, bz, cx, cy, cz):
    abx = bx - ax
    aby = by - ay
    abz = bz - az
    acx = cx - ax
    acy = cy - ay
    acz = cz - az

    apx = px - ax
    apy = py - ay
    apz = pz - az
    d1 = abx * apx + aby * apy + abz * apz
    d2 = acx * apx + acy * apy + acz * apz

    bpx = px - bx
    bpy = py - by
    bpz = pz - bz
    d3 = abx * bpx + aby * bpy + abz * bpz
    d4 = acx * bpx + acy * bpy + acz * bpz

    cpx = px - cx
    cpy = py - cy
    cpz = pz - cz
    d5 = abx * cpx + aby * cpy + abz * cpz
    d6 = acx * cpx + acy * cpy + acz * cpz

    vc = d1 * d4 - d3 * d2
    vb = d5 * d2 - d1 * d6
    va = d3 * d6 - d5 * d4

    v_ab = _safe_div(d1, d1 - d3)
    w_ac = _safe_div(d2, d2 - d6)
    w_bc = _safe_div(d4 - d3, (d4 - d3) + (d5 - d6))
    denom = _safe_div(jnp.ones_like(va), va + vb + vc)
    v_in = vb * denom
    w_in = vc * denom

    cond_a = (d1 <= 0) & (d2 <= 0)
    cond_b = (d3 >= 0) & (d4 <= d3)
    cond_ab = (vc <= 0) & (d1 >= 0) & (d3 <= 0)
    cond_c = (d6 >= 0) & (d5 <= d6)
    cond_ac = (vb <= 0) & (d2 >= 0) & (d6 <= 0)
    cond_bc = (va <= 0) & ((d4 - d3) >= 0) & ((d5 - d6) >= 0)

    u = 1.0 - v_in - w_in
    v = v_in
    w = w_in

    zero = jnp.zeros_like(u)
    one = jnp.ones_like(u)
    u = jnp.where(cond_bc, zero, u)
    v = jnp.where(cond_bc, 1.0 - w_bc, v)
    w = jnp.where(cond_bc, w_bc, w)
    u = jnp.where(cond_ac, 1.0 - w_ac, u)
    v = jnp.where(cond_ac, zero, v)
    w = jnp.where(cond_ac, w_ac, w)
    u = jnp.where(cond_c, zero, u)
    v = jnp.where(cond_c, zero, v)
    w = jnp.where(cond_c, one, w)
    u = jnp.where(cond_ab, 1.0 - v_ab, u)
    v = jnp.where(cond_ab, v_ab, v)
    w = jnp.where(cond_ab, zero, w)
    u = jnp.where(cond_b, zero, u)
    v = jnp.where(cond_b, one, v)
    w = jnp.where(cond_b, zero, w)
    u = jnp.where(cond_a, one, u)
    v = jnp.where(cond_a, zero, v)
    w = jnp.where(cond_a, zero, w)

    clx = u * ax + v * bx + w * cx
    cly = u * ay + v * by + w * cy
    clz = u * az + v * bz + w * cz
    dx = px - clx
    dy = py - cly
    dz = pz - clz
    dist = dx * dx + dy * dy + dz * dz
    return dist, u, v, w, clx, cly, clz


def _tc_kernel(tri_ref, pts_ref, d_ref, cp_ref, idx_ref, bc_ref, *, F, FT, CQ):
    px = pts_ref[:, 0:1]
    py = pts_ref[:, 1:2]
    pz = pts_ref[:, 2:3]

    best_d = jnp.full((CQ, 1), jnp.inf, jnp.float32)
    best_idx = jnp.zeros((CQ, 1), jnp.int32)
    best_u = jnp.zeros((CQ, 1), jnp.float32)
    best_v = jnp.zeros((CQ, 1), jnp.float32)
    best_w = jnp.zeros((CQ, 1), jnp.float32)
    best_cx = jnp.zeros((CQ, 1), jnp.float32)
    best_cy = jnp.zeros((CQ, 1), jnp.float32)
    best_cz = jnp.zeros((CQ, 1), jnp.float32)

    lane = lax.broadcasted_iota(jnp.int32, (1, FT), 1)

    for t in range(F // FT):
        s = slice(t * FT, (t + 1) * FT)
        dist, u, v, w, clx, cly, clz = _ericson(
            px, py, pz,
            tri_ref[0:1, s], tri_ref[1:2, s], tri_ref[2:3, s],
            tri_ref[3:4, s], tri_ref[4:5, s], tri_ref[5:6, s],
            tri_ref[6:7, s], tri_ref[7:8, s], tri_ref[8:9, s],
        )

        dmin = jnp.min(dist, axis=1, keepdims=True)
        at_min = dist == dmin
        idx_t = jnp.min(jnp.where(at_min, lane, _BIG_I32), axis=1, keepdims=True)
        sel = lane == idx_t

        def pick(val):
            return jnp.sum(jnp.where(sel, val, 0.0), axis=1, keepdims=True)

        u_t, v_t, w_t = pick(u), pick(v), pick(w)
        cx_t, cy_t, cz_t = pick(clx), pick(cly), pick(clz)

        better = dmin < best_d
        best_d = jnp.where(better, dmin, best_d)
        best_idx = jnp.where(better, idx_t + t * FT, best_idx)
        best_u = jnp.where(better, u_t, best_u)
        best_v = jnp.where(better, v_t, best_v)
        best_w = jnp.where(better, w_t, best_w)
        best_cx = jnp.where(better, cx_t, best_cx)
        best_cy = jnp.where(better, cy_t, best_cy)
        best_cz = jnp.where(better, cz_t, best_cz)

    d_ref[:, :] = best_d
    idx_ref[:, :] = best_idx
    cp_ref[:, :] = jnp.concatenate([best_cx, best_cy, best_cz], axis=1)
    bc_ref[:, :] = jnp.concatenate([best_u, best_v, best_w], axis=1)


def _tc_query(tri9, pts, CQ=256, FT=512):
    F = tri9.shape[1]
    Q = pts.shape[0]
    out_shape = [
        jax.ShapeDtypeStruct((Q, 1), jnp.float32),
        jax.ShapeDtypeStruct((Q, 3), jnp.float32),
        jax.ShapeDtypeStruct((Q, 1), jnp.int32),
        jax.ShapeDtypeStruct((Q, 3), jnp.float32),
    ]
    grid = (Q // CQ,)
    d, cp, idx, bc = pl.pallas_call(
        functools.partial(_tc_kernel, F=F, FT=FT, CQ=CQ),
        grid=grid,
        in_specs=[
            pl.BlockSpec((9, F), lambda i: (0, 0)),
            pl.BlockSpec((CQ, 3), lambda i: (i, 0)),
        ],
        out_specs=[
            pl.BlockSpec((CQ, 1), lambda i: (i, 0)),
            pl.BlockSpec((CQ, 3), lambda i: (i, 0)),
            pl.BlockSpec((CQ, 1), lambda i: (i, 0)),
            pl.BlockSpec((CQ, 3), lambda i: (i, 0)),
        ],
        out_shape=out_shape,
    )(tri9, pts)
    return d[:, 0], cp, idx[:, 0], bc


def _take16(x, perm):
    return lax.gather(
        x, perm[:, None],
        dimension_numbers=lax.GatherDimensionNumbers(
            offset_dims=(), collapsed_slice_dims=(0,), start_index_map=(0,)),
        slice_sizes=(1,),
        mode=lax.GatherScatterMode.PROMISE_IN_BOUNDS,
    )


def _sc_body(tri_hbm, px_hbm, py_hbm, pz_hbm,
             d_hbm, i_hbm, u_hbm, v_hbm, w_hbm, cx_hbm, cy_hbm, cz_hbm,
             tri_v, px_v, py_v, pz_v,
             do_v, io_v, uo_v, vo_v, wo_v, cxo_v, cyo_v, czo_v,
             *, F, CH):
    wid = lax.axis_index("s") * 2 + lax.axis_index("c")
    base = wid * CH

    pltpu.sync_copy(tri_hbm, tri_v)
    pltpu.sync_copy(px_hbm.at[pl.ds(base, CH)], px_v)
    pltpu.sync_copy(py_hbm.at[pl.ds(base, CH)], py_v)
    pltpu.sync_copy(pz_hbm.at[pl.ds(base, CH)], pz_v)

    lane = lax.iota(jnp.int32, 16)

    def point_body(i, carry):
        iv = jnp.full((16,), i, jnp.int32)
        px = plsc.load_gather(px_v, [iv])
        py = plsc.load_gather(py_v, [iv])
        pz = plsc.load_gather(pz_v, [iv])

        init = (
            jnp.full((16,), jnp.inf, jnp.float32),
            jnp.zeros((16,), jnp.int32),
            jnp.zeros((16,), jnp.float32),
            jnp.zeros((16,), jnp.float32),
            jnp.zeros((16,), jnp.float32),
            jnp.zeros((16,), jnp.float32),
            jnp.zeros((16,), jnp.float32),
            jnp.zeros((16,), jnp.float32),
        )

        def tri_body(t, c):
            bd, bi, bu, bv, bw, bx_, by_, bz_ = c
            o = t * 16
            dist, u, v, w, clx, cly, clz = _ericson(
                px, py, pz,
                tri_v[pl.ds(o, 16)],
                tri_v[pl.ds(F + o, 16)],
                tri_v[pl.ds(2 * F + o, 16)],
                tri_v[pl.ds(3 * F + o, 16)],
                tri_v[pl.ds(4 * F + o, 16)],
                tri_v[pl.ds(5 * F + o, 16)],
                tri_v[pl.ds(6 * F + o, 16)],
                tri_v[pl.ds(7 * F + o, 16)],
                tri_v[pl.ds(8 * F + o, 16)],
            )
            tidx = lane + o
            better = dist < bd
            return (
                jnp.where(better, dist, bd),
                jnp.where(better, tidx, bi),
                jnp.where(better, u, bu),
                jnp.where(better, v, bv),
                jnp.where(better, w, bw),
                jnp.where(better, clx, bx_),
                jnp.where(better, cly, by_),
                jnp.where(better, clz, bz_),
            )

        bd, bi, bu, bv, bw, bcx, bcy, bcz = lax.fori_loop(
            0, F // 16, tri_body, init)

        for s in (8, 4, 2, 1):
            perm = lane ^ s
            d2 = _take16(bd, perm)
            i2 = _take16(bi, perm)
            u2 = _take16(bu, perm)
            v2 = _take16(bv, perm)
            w2 = _take16(bw, perm)
            x2 = _take16(bcx, perm)
            y2 = _take16(bcy, perm)
            z2 = _take16(bcz, perm)
            better = (d2 < bd) | ((d2 == bd) & (i2 < bi))
            bd = jnp.where(better, d2, bd)
            bi = jnp.where(better, i2, bi)
            bu = jnp.where(better, u2, bu)
            bv = jnp.where(better, v2, bv)
            bw = jnp.where(better, w2, bw)
            bcx = jnp.where(better, x2, bcx)
            bcy = jnp.where(better, y2, bcy)
            bcz = jnp.where(better, z2, bcz)

        lane0 = lane == jnp.zeros((16,), jnp.int32)
        plsc.store_scatter(do_v, [iv], bd, mask=lane0)
        plsc.store_scatter(io_v, [iv], bi, mask=lane0)
        plsc.store_scatter(uo_v, [iv], bu, mask=lane0)
        plsc.store_scatter(vo_v, [iv], bv, mask=lane0)
        plsc.store_scatter(wo_v, [iv], bw, mask=lane0)
        plsc.store_scatter(cxo_v, [iv], bcx, mask=lane0)
        plsc.store_scatter(cyo_v, [iv], bcy, mask=lane0)
        plsc.store_scatter(czo_v, [iv], bcz, mask=lane0)
        return carry

    lax.fori_loop(0, CH, point_body, 0)

    pltpu.sync_copy(do_v, d_hbm.at[pl.ds(base, CH)])
    pltpu.sync_copy(io_v, i_hbm.at[pl.ds(base, CH)])
    pltpu.sync_copy(uo_v, u_hbm.at[pl.ds(base, CH)])
    pltpu.sync_copy(vo_v, v_hbm.at[pl.ds(base, CH)])
    pltpu.sync_copy(wo_v, w_hbm.at[pl.ds(base, CH)])
    pltpu.sync_copy(cxo_v, cx_hbm.at[pl.ds(base, CH)])
    pltpu.sync_copy(cyo_v, cy_hbm.at[pl.ds(base, CH)])
    pltpu.sync_copy(czo_v, cz_hbm.at[pl.ds(base, CH)])


def _sc_query(tri9, pts):
    F = tri9.shape[1]
    Q = pts.shape[0]
    CH = Q // _NW
    tri_flat = tri9.reshape(9 * F)
    px, py, pz = pts[:, 0], pts[:, 1], pts[:, 2]

    f32 = jnp.float32
    call = pl.kernel(
        functools.partial(_sc_body, F=F, CH=CH),
        out_type=[
            jax.ShapeDtypeStruct((Q,), f32),
            jax.ShapeDtypeStruct((Q,), jnp.int32),
            jax.ShapeDtypeStruct((Q,), f32),
            jax.ShapeDtypeStruct((Q,), f32),
            jax.ShapeDtypeStruct((Q,), f32),
            jax.ShapeDtypeStruct((Q,), f32),
            jax.ShapeDtypeStruct((Q,), f32),
            jax.ShapeDtypeStruct((Q,), f32),
        ],
        mesh=plsc.VectorSubcoreMesh(core_axis_name="c", subcore_axis_name="s", num_cores=2),
        compiler_params=pltpu.CompilerParams(needs_layout_passes=False),
        scratch_types=[
            pltpu.VMEM((9 * F,), f32),
            pltpu.VMEM((CH,), f32),
            pltpu.VMEM((CH,), f32),
            pltpu.VMEM((CH,), f32),
            pltpu.VMEM((CH,), f32),
            pltpu.VMEM((CH,), jnp.int32),
            pltpu.VMEM((CH,), f32),
            pltpu.VMEM((CH,), f32),
            pltpu.VMEM((CH,), f32),
            pltpu.VMEM((CH,), f32),
            pltpu.VMEM((CH,), f32),
            pltpu.VMEM((CH,), f32),
        ],
    )
    d, idx, u, v, w, cx, cy, cz = call(tri_flat, px, py, pz)
    cp = jnp.stack([cx, cy, cz], axis=1)
    bc = jnp.stack([u, v, w], axis=1)
    return d, cp, idx, bc


def _query_one(tris, pts):
    F = tris.shape[0]
    Q = pts.shape[0]
    tri9 = tris.reshape(F, 9).T

    q_sc = _Q_SC
    q_tc = Q - q_sc
    d_tc, cp_tc, idx_tc, bc_tc = _tc_query(tri9, pts[:q_tc])
    d_sc, cp_sc, idx_sc, bc_sc = _sc_query(tri9, pts[q_tc:])
    d = jnp.concatenate([d_tc, d_sc])
    cp = jnp.concatenate([cp_tc, cp_sc])
    idx = jnp.concatenate([idx_tc, idx_sc])
    bc = jnp.concatenate([bc_tc, bc_sc])
    return d, cp, idx, bc


def kernel(triangles, points):
    B = triangles.shape[0]
    ds, cps, idxs, bcs = [], [], [], []
    for b in range(B):
        d, cp, idx, bc = _query_one(triangles[b], points[b])
        ds.append(d)
        cps.append(cp)
        idxs.append(idx)
        bcs.append(bc)
    distances = jnp.stack(ds)
    closest_points = jnp.stack(cps)
    closest_faces = jnp.stack(idxs).astype(jnp.int64)
    closest_bcs = jnp.stack(bcs)
    return distances, closest_points, closest_faces, closest_bcs


# speedup vs baseline: 76.1430x; 1.0205x over previous
import functools

import jax
import jax.numpy as jnp
from jax import lax
from jax.experimental import pallas as pl
from jax.experimental.pallas import tpu as pltpu
from jax.experimental.pallas import tpu_sc as plsc

_EPS = 1e-12
_BIG_I32 = 2**31 - 1

_Q_SC = 2304
_NW = 32


def _safe_div(a, b):
    b_safe = jnp.where(jnp.abs(b) < _EPS, jnp.where(b < 0, -_EPS, _EPS), b)
    return a / b_safe


def _ericson(px, py, pz, ax, ay, az, bx, by, bz, cx, cy, cz):
    abx = bx - ax
    aby = by - ay
    abz = bz - az
    acx = cx - ax
    acy = cy - ay
    acz = cz - az

    apx = px - ax
    apy = py - ay
    apz = pz - az
    d1 = abx * apx + aby * apy + abz * apz
    d2 = acx * apx + acy * apy + acz * apz

    bpx = px - bx
    bpy = py - by
    bpz = pz - bz
    d3 = abx * bpx + aby * bpy + abz * bpz
    d4 = acx * bpx + acy * bpy + acz * bpz

    cpx = px - cx
    cpy = py - cy
    cpz = pz - cz
    d5 = abx * cpx + aby * cpy + abz * cpz
    d6 = acx * cpx + acy * cpy + acz * cpz

    vc = d1 * d4 - d3 * d2
    vb = d5 * d2 - d1 * d6
    va = d3 * d6 - d5 * d4

    v_ab = _safe_div(d1, d1 - d3)
    w_ac = _safe_div(d2, d2 - d6)
    w_bc = _safe_div(d4 - d3, (d4 - d3) + (d5 - d6))
    denom = _safe_div(jnp.ones_like(va), va + vb + vc)
    v_in = vb * denom
    w_in = vc * denom

    cond_a = (d1 <= 0) & (d2 <= 0)
    cond_b = (d3 >= 0) & (d4 <= d3)
    cond_ab = (vc <= 0) & (d1 >= 0) & (d3 <= 0)
    cond_c = (d6 >= 0) & (d5 <= d6)
    cond_ac = (vb <= 0) & (d2 >= 0) & (d6 <= 0)
    cond_bc = (va <= 0) & ((d4 - d3) >= 0) & ((d5 - d6) >= 0)

    u = 1.0 - v_in - w_in
    v = v_in
    w = w_in

    zero = jnp.zeros_like(u)
    one = jnp.ones_like(u)
    u = jnp.where(cond_bc, zero, u)
    v = jnp.where(cond_bc, 1.0 - w_bc, v)
    w = jnp.where(cond_bc, w_bc, w)
    u = jnp.where(cond_ac, 1.0 - w_ac, u)
    v = jnp.where(cond_ac, zero, v)
    w = jnp.where(cond_ac, w_ac, w)
    u = jnp.where(cond_c, zero, u)
    v = jnp.where(cond_c, zero, v)
    w = jnp.where(cond_c, one, w)
    u = jnp.where(cond_ab, 1.0 - v_ab, u)
    v = jnp.where(cond_ab, v_ab, v)
    w = jnp.where(cond_ab, zero, w)
    u = jnp.where(cond_b, zero, u)
    v = jnp.where(cond_b, one, v)
    w = jnp.where(cond_b, zero, w)
    u = jnp.where(cond_a, one, u)
    v = jnp.where(cond_a, zero, v)
    w = jnp.where(cond_a, zero, w)

    clx = u * ax + v * bx + w * cx
    cly = u * ay + v * by + w * cy
    clz = u * az + v * bz + w * cz
    dx = px - clx
    dy = py - cly
    dz = pz - clz
    dist = dx * dx + dy * dy + dz * dz
    return dist, u, v, w, clx, cly, clz


def _tc_kernel(tri_ref, pts_ref, d_ref, cp_ref, idx_ref, bc_ref, *, F, FT, CQ):
    px = pts_ref[:, 0:1]
    py = pts_ref[:, 1:2]
    pz = pts_ref[:, 2:3]

    best_d = jnp.full((CQ, 1), jnp.inf, jnp.float32)
    best_idx = jnp.zeros((CQ, 1), jnp.int32)
    best_u = jnp.zeros((CQ, 1), jnp.float32)
    best_v = jnp.zeros((CQ, 1), jnp.float32)
    best_w = jnp.zeros((CQ, 1), jnp.float32)
    best_cx = jnp.zeros((CQ, 1), jnp.float32)
    best_cy = jnp.zeros((CQ, 1), jnp.float32)
    best_cz = jnp.zeros((CQ, 1), jnp.float32)

    lane = lax.broadcasted_iota(jnp.int32, (1, FT), 1)

    for t in range(F // FT):
        s = slice(t * FT, (t + 1) * FT)
        dist, u, v, w, clx, cly, clz = _ericson(
            px, py, pz,
            tri_ref[0:1, s], tri_ref[1:2, s], tri_ref[2:3, s],
            tri_ref[3:4, s], tri_ref[4:5, s], tri_ref[5:6, s],
            tri_ref[6:7, s], tri_ref[7:8, s], tri_ref[8:9, s],
        )

        dmin = jnp.min(dist, axis=1, keepdims=True)
        at_min = dist == dmin
        idx_t = jnp.min(jnp.where(at_min, lane, _BIG_I32), axis=1, keepdims=True)
        sel = lane == idx_t

        def pick(val):
            return jnp.sum(jnp.where(sel, val, 0.0), axis=1, keepdims=True)

        u_t, v_t, w_t = pick(u), pick(v), pick(w)
        cx_t, cy_t, cz_t = pick(clx), pick(cly), pick(clz)

        better = dmin < best_d
        best_d = jnp.where(better, dmin, best_d)
        best_idx = jnp.where(better, idx_t + t * FT, best_idx)
        best_u = jnp.where(better, u_t, best_u)
        best_v = jnp.where(better, v_t, best_v)
        best_w = jnp.where(better, w_t, best_w)
        best_cx = jnp.where(better, cx_t, best_cx)
        best_cy = jnp.where(better, cy_t, best_cy)
        best_cz = jnp.where(better, cz_t, best_cz)

    d_ref[:, :] = best_d
    idx_ref[:, :] = best_idx
    cp_ref[:, :] = jnp.concatenate([best_cx, best_cy, best_cz], axis=1)
    bc_ref[:, :] = jnp.concatenate([best_u, best_v, best_w], axis=1)


def _tc_query(tri9, pts, CQ=256, FT=512):
    F = tri9.shape[1]
    Q = pts.shape[0]
    out_shape = [
        jax.ShapeDtypeStruct((Q, 1), jnp.float32),
        jax.ShapeDtypeStruct((Q, 3), jnp.float32),
        jax.ShapeDtypeStruct((Q, 1), jnp.int32),
        jax.ShapeDtypeStruct((Q, 3), jnp.float32),
    ]
    grid = (Q // CQ,)
    d, cp, idx, bc = pl.pallas_call(
        functools.partial(_tc_kernel, F=F, FT=FT, CQ=CQ),
        grid=grid,
        in_specs=[
            pl.BlockSpec((9, F), lambda i: (0, 0)),
            pl.BlockSpec((CQ, 3), lambda i: (i, 0)),
        ],
        out_specs=[
            pl.BlockSpec((CQ, 1), lambda i: (i, 0)),
            pl.BlockSpec((CQ, 3), lambda i: (i, 0)),
            pl.BlockSpec((CQ, 1), lambda i: (i, 0)),
            pl.BlockSpec((CQ, 3), lambda i: (i, 0)),
        ],
        out_shape=out_shape,
    )(tri9, pts)
    return d[:, 0], cp, idx[:, 0], bc


def _take16(x, perm):
    return lax.gather(
        x, perm[:, None],
        dimension_numbers=lax.GatherDimensionNumbers(
            offset_dims=(), collapsed_slice_dims=(0,), start_index_map=(0,)),
        slice_sizes=(1,),
        mode=lax.GatherScatterMode.PROMISE_IN_BOUNDS,
    )


def _sc_body(tri_hbm, px_hbm, py_hbm, pz_hbm,
             d_hbm, i_hbm, u_hbm, v_hbm, w_hbm, cx_hbm, cy_hbm, cz_hbm,
             tri_v, px_v, py_v, pz_v,
             do_v, io_v, uo_v, vo_v, wo_v, cxo_v, cyo_v, czo_v,
             *, F, CH):
    wid = lax.axis_index("s") * 2 + lax.axis_index("c")
    base = wid * CH

    pltpu.sync_copy(tri_hbm, tri_v)
    pltpu.sync_copy(px_hbm.at[pl.ds(base, CH)], px_v)
    pltpu.sync_copy(py_hbm.at[pl.ds(base, CH)], py_v)
    pltpu.sync_copy(pz_hbm.at[pl.ds(base, CH)], pz_v)

    lane = lax.iota(jnp.int32, 16)

    def point_body(i, carry):
        iv = jnp.full((16,), i, jnp.int32)
        px = plsc.load_gather(px_v, [iv])
        py = plsc.load_gather(py_v, [iv])
        pz = plsc.load_gather(pz_v, [iv])

        init = (
            jnp.full((16,), jnp.inf, jnp.float32),
            jnp.zeros((16,), jnp.int32),
            jnp.zeros((16,), jnp.float32),
            jnp.zeros((16,), jnp.float32),
            jnp.zeros((16,), jnp.float32),
            jnp.zeros((16,), jnp.float32),
            jnp.zeros((16,), jnp.float32),
            jnp.zeros((16,), jnp.float32),
        )

        def tri_body(t, c):
            bd, bi, bu, bv, bw, bx_, by_, bz_ = c
            o = t * 16
            dist, u, v, w, clx, cly, clz = _ericson(
                px, py, pz,
                tri_v[pl.ds(o, 16)],
                tri_v[pl.ds(F + o, 16)],
                tri_v[pl.ds(2 * F + o, 16)],
                tri_v[pl.ds(3 * F + o, 16)],
                tri_v[pl.ds(4 * F + o, 16)],
                tri_v[pl.ds(5 * F + o, 16)],
                tri_v[pl.ds(6 * F + o, 16)],
                tri_v[pl.ds(7 * F + o, 16)],
                tri_v[pl.ds(8 * F + o, 16)],
            )
            tidx = lane + o
            better = dist < bd
            return (
                jnp.where(better, dist, bd),
                jnp.where(better, tidx, bi),
                jnp.where(better, u, bu),
                jnp.where(better, v, bv),
                jnp.where(better, w, bw),
                jnp.where(better, clx, bx_),
                jnp.where(better, cly, by_),
                jnp.where(better, clz, bz_),
            )

        bd, bi, bu, bv, bw, bcx, bcy, bcz = lax.fori_loop(
            0, F // 16, tri_body, init)

        for s in (8, 4, 2, 1):
            perm = lane ^ s
            d2 = _take16(bd, perm)
            i2 = _take16(bi, perm)
            u2 = _take16(bu, perm)
            v2 = _take16(bv, perm)
            w2 = _take16(bw, perm)
            x2 = _take16(bcx, perm)
            y2 = _take16(bcy, perm)
            z2 = _take16(bcz, perm)
            better = (d2 < bd) | ((d2 == bd) & (i2 < bi))
            bd = jnp.where(better, d2, bd)
            bi = jnp.where(better, i2, bi)
            bu = jnp.where(better, u2, bu)
            bv = jnp.where(better, v2, bv)
            bw = jnp.where(better, w2, bw)
            bcx = jnp.where(better, x2, bcx)
            bcy = jnp.where(better, y2, bcy)
            bcz = jnp.where(better, z2, bcz)

        lane0 = lane == jnp.zeros((16,), jnp.int32)
        plsc.store_scatter(do_v, [iv], bd, mask=lane0)
        plsc.store_scatter(io_v, [iv], bi, mask=lane0)
        plsc.store_scatter(uo_v, [iv], bu, mask=lane0)
        plsc.store_scatter(vo_v, [iv], bv, mask=lane0)
        plsc.store_scatter(wo_v, [iv], bw, mask=lane0)
        plsc.store_scatter(cxo_v, [iv], bcx, mask=lane0)
        plsc.store_scatter(cyo_v, [iv], bcy, mask=lane0)
        plsc.store_scatter(czo_v, [iv], bcz, mask=lane0)
        return carry

    lax.fori_loop(0, CH, point_body, 0)

    pltpu.sync_copy(do_v, d_hbm.at[pl.ds(base, CH)])
    pltpu.sync_copy(io_v, i_hbm.at[pl.ds(base, CH)])
    pltpu.sync_copy(uo_v, u_hbm.at[pl.ds(base, CH)])
    pltpu.sync_copy(vo_v, v_hbm.at[pl.ds(base, CH)])
    pltpu.sync_copy(wo_v, w_hbm.at[pl.ds(base, CH)])
    pltpu.sync_copy(cxo_v, cx_hbm.at[pl.ds(base, CH)])
    pltpu.sync_copy(cyo_v, cy_hbm.at[pl.ds(base, CH)])
    pltpu.sync_copy(czo_v, cz_hbm.at[pl.ds(base, CH)])


def _sc_query(tri9, pts):
    F = tri9.shape[1]
    Q = pts.shape[0]
    CH = Q // _NW
    tri_flat = tri9.reshape(9 * F)
    px, py, pz = pts[:, 0], pts[:, 1], pts[:, 2]

    f32 = jnp.float32
    call = pl.kernel(
        functools.partial(_sc_body, F=F, CH=CH),
        out_type=[
            jax.ShapeDtypeStruct((Q,), f32),
            jax.ShapeDtypeStruct((Q,), jnp.int32),
            jax.ShapeDtypeStruct((Q,), f32),
            jax.ShapeDtypeStruct((Q,), f32),
            jax.ShapeDtypeStruct((Q,), f32),
            jax.ShapeDtypeStruct((Q,), f32),
            jax.ShapeDtypeStruct((Q,), f32),
            jax.ShapeDtypeStruct((Q,), f32),
        ],
        mesh=plsc.VectorSubcoreMesh(core_axis_name="c", subcore_axis_name="s", num_cores=2),
        compiler_params=pltpu.CompilerParams(needs_layout_passes=False),
        scratch_types=[
            pltpu.VMEM((9 * F,), f32),
            pltpu.VMEM((CH,), f32),
            pltpu.VMEM((CH,), f32),
            pltpu.VMEM((CH,), f32),
            pltpu.VMEM((CH,), f32),
            pltpu.VMEM((CH,), jnp.int32),
            pltpu.VMEM((CH,), f32),
            pltpu.VMEM((CH,), f32),
            pltpu.VMEM((CH,), f32),
            pltpu.VMEM((CH,), f32),
            pltpu.VMEM((CH,), f32),
            pltpu.VMEM((CH,), f32),
        ],
    )
    d, idx, u, v, w, cx, cy, cz = call(tri_flat, px, py, pz)
    cp = jnp.stack([cx, cy, cz], axis=1)
    bc = jnp.stack([u, v, w], axis=1)
    return d, cp, idx, bc


def _query_one(tris, pts):
    F = tris.shape[0]
    Q = pts.shape[0]
    tri9 = tris.reshape(F, 9).T

    q_sc = _Q_SC
    q_tc = Q - q_sc
    d_tc, cp_tc, idx_tc, bc_tc = _tc_query(tri9, pts[:q_tc])
    d_sc, cp_sc, idx_sc, bc_sc = _sc_query(tri9, pts[q_tc:])
    d = jnp.concatenate([d_tc, d_sc])
    cp = jnp.concatenate([cp_tc, cp_sc])
    idx = jnp.concatenate([idx_tc, idx_sc])
    bc = jnp.concatenate([bc_tc, bc_sc])
    return d, cp, idx, bc


def kernel(triangles, points):
    B = triangles.shape[0]
    ds, cps, idxs, bcs = [], [], [], []
    for b in range(B):
        d, cp, idx, bc = _query_one(triangles[b], points[b])
        ds.append(d)
        cps.append(cp)
        idxs.append(idx)
        bcs.append(bc)
    distances = jnp.stack(ds)
    closest_points = jnp.stack(cps)
    closest_faces = jnp.stack(idxs).astype(jnp.int64)
    closest_bcs = jnp.stack(bcs)
    return distances, closest_points, closest_faces, closest_bcs
